# baseline (speedup 1.0000x reference)
import jax
import jax.numpy as jnp
from jax.experimental import pallas as pl
from jax.experimental.pallas import tpu as pltpu


def _sage_block_kernel(a1_ref, a2_ref, h_ref, wt_ref, b_ref, out_ref):
    half = a1_ref.shape[0]
    y1 = jnp.dot(a1_ref[...], h_ref[...], preferred_element_type=jnp.float32)
    out_ref[:half, :] = (
        jnp.dot(y1, wt_ref[...], preferred_element_type=jnp.float32) + b_ref[...]
    )
    y2 = jnp.dot(a2_ref[...], h_ref[...], preferred_element_type=jnp.float32)
    out_ref[half:, :] = (
        jnp.dot(y2, wt_ref[...], preferred_element_type=jnp.float32) + b_ref[...]
    )


def kernel(adj, h, W, bias):
    n, d_in = h.shape
    d_out = W.shape[0]

    sub_rows = 200
    step_rows = 2 * sub_rows

    wt = W.T
    bias2d = bias.reshape(1, d_out)

    out = pl.pallas_call(
        _sage_block_kernel,
        grid=(n // step_rows,),
        in_specs=[
            pl.BlockSpec((sub_rows, n), lambda i: (2 * i, 0)),
            pl.BlockSpec((sub_rows, n), lambda i: (2 * i + 1, 0)),
            pl.BlockSpec((n, d_in), lambda i: (0, 0)),
            pl.BlockSpec((d_in, d_out), lambda i: (0, 0)),
            pl.BlockSpec((1, d_out), lambda i: (0, 0)),
        ],
        out_specs=pl.BlockSpec((step_rows, d_out), lambda i: (i, 0)),
        out_shape=jax.ShapeDtypeStruct((n, d_out), jnp.float32),
        compiler_params=pltpu.CompilerParams(
            dimension_semantics=("arbitrary",),
        ),
    )(adj, adj, h, wt, bias2d)
    return out


# speedup vs baseline: 1.0406x; 1.0406x over previous
import jax
import jax.numpy as jnp
from jax.experimental import pallas as pl
from jax.experimental.pallas import tpu as pltpu

_BLOCK = 400
_NBUF = 3


def _sage_pipeline_kernel(adj_hbm, h_ref, wt_ref, b_ref, out_ref, abuf, sems):
    n = adj_hbm.shape[0]
    nblk = n // _BLOCK

    def copy(i):
        return pltpu.make_async_copy(
            adj_hbm.at[pl.ds(i * _BLOCK, _BLOCK), :],
            abuf.at[i % _NBUF],
            sems.at[i % _NBUF],
        )

    for s in range(_NBUF):
        copy(s).start()

    for i in range(nblk):
        copy(i).wait()
        y = jnp.dot(
            abuf[i % _NBUF], h_ref[...], preferred_element_type=jnp.float32
        )
        out_ref[i * _BLOCK : (i + 1) * _BLOCK, :] = (
            jnp.dot(y, wt_ref[...], preferred_element_type=jnp.float32)
            + b_ref[...]
        )
        if i + _NBUF < nblk:
            copy(i + _NBUF).start()


def kernel(adj, h, W, bias):
    n, d_in = h.shape
    d_out = W.shape[0]

    wt = W.T
    bias2d = bias.reshape(1, d_out)

    out = pl.pallas_call(
        _sage_pipeline_kernel,
        in_specs=[
            pl.BlockSpec(memory_space=pltpu.MemorySpace.HBM),
            pl.BlockSpec(memory_space=pltpu.MemorySpace.VMEM),
            pl.BlockSpec(memory_space=pltpu.MemorySpace.VMEM),
            pl.BlockSpec(memory_space=pltpu.MemorySpace.VMEM),
        ],
        out_specs=pl.BlockSpec(memory_space=pltpu.MemorySpace.VMEM),
        out_shape=jax.ShapeDtypeStruct((n, d_out), jnp.float32),
        scratch_shapes=[
            pltpu.VMEM((_NBUF, _BLOCK, n), jnp.float32),
            pltpu.SemaphoreType.DMA((_NBUF,)),
        ],
    )(adj, h, wt, bias2d)
    return out


# speedup vs baseline: 1.0496x; 1.0087x over previous
import jax
import jax.numpy as jnp
from jax.experimental import pallas as pl
from jax.experimental.pallas import tpu as pltpu


def _sage_block_kernel(adj_ref, h_ref, wt_ref, b_ref, out_ref):
    y = jnp.dot(adj_ref[...], h_ref[...], preferred_element_type=jnp.float32)
    out_ref[...] = (
        jnp.dot(y, wt_ref[...], preferred_element_type=jnp.float32) + b_ref[...]
    )


def kernel(adj, h, W, bias):
    n, d_in = h.shape
    d_out = W.shape[0]

    block_rows = 640

    wt = W.T
    bias2d = bias.reshape(1, d_out)

    out = pl.pallas_call(
        _sage_block_kernel,
        grid=((n + block_rows - 1) // block_rows,),
        in_specs=[
            pl.BlockSpec((block_rows, n), lambda i: (i, 0)),
            pl.BlockSpec((n, d_in), lambda i: (0, 0)),
            pl.BlockSpec((d_in, d_out), lambda i: (0, 0)),
            pl.BlockSpec((1, d_out), lambda i: (0, 0)),
        ],
        out_specs=pl.BlockSpec((block_rows, d_out), lambda i: (i, 0)),
        out_shape=jax.ShapeDtypeStruct((n, d_out), jnp.float32),
        compiler_params=pltpu.CompilerParams(
            dimension_semantics=("arbitrary",),
        ),
    )(adj, h, wt, bias2d)
    return out


# speedup vs baseline: 1.0713x; 1.0206x over previous
import jax
import jax.numpy as jnp
from jax.experimental import pallas as pl
from jax.experimental.pallas import tpu as pltpu


def _sage_block_kernel(adj_ref, h_ref, wt_ref, b_ref, out_ref):
    y = jnp.dot(adj_ref[...], h_ref[...], preferred_element_type=jnp.float32)
    out_ref[...] = (
        jnp.dot(y, wt_ref[...], preferred_element_type=jnp.float32) + b_ref[...]
    )


def kernel(adj, h, W, bias):
    n, d_in = h.shape
    d_out = W.shape[0]

    block_rows = 400

    wt = W.T
    bias2d = bias.reshape(1, d_out)

    out = pl.pallas_call(
        _sage_block_kernel,
        grid=((n + block_rows - 1) // block_rows,),
        in_specs=[
            pl.BlockSpec((block_rows, n), lambda i: (i, 0)),
            pl.BlockSpec((n, d_in), lambda i: (0, 0)),
            pl.BlockSpec((d_in, d_out), lambda i: (0, 0)),
            pl.BlockSpec((1, d_out), lambda i: (0, 0)),
        ],
        out_specs=pl.BlockSpec((block_rows, d_out), lambda i: (i, 0)),
        out_shape=jax.ShapeDtypeStruct((n, d_out), jnp.float32),
        compiler_params=pltpu.CompilerParams(
            dimension_semantics=("arbitrary",),
        ),
    )(adj, h, wt, bias2d)
    return out
